# Initial kernel scaffold; baseline (speedup 1.0000x reference)
import jax
import jax.numpy as jnp
from jax.experimental import pallas as pl


def kernel(x):
    raise NotImplementedError("write your pallas kernel here")


# speedup vs baseline: 35.8699x; 35.8699x over previous
import functools

import numpy as np
import jax
import jax.numpy as jnp
from jax import lax
from jax.experimental import pallas as pl
from jax.experimental.pallas import tpu as pltpu
from jax.experimental.pallas import tpu_sc as plsc

N = 4096
NPAIRS = N * (N - 1) // 2
FLAT = 2 * NPAIRS
NC, NS, L = 2, 16, 16
NW = NC * NS
CPW = 10
NCHUNKS = NW * CPW
CHUNK = FLAT // NCHUNKS
CPAIRS = CHUNK // 2
RECW = 16
MAXIT = 232
ROWW = (1 + MAXIT) * RECW
XPAD = N + L


def _build_table() -> np.ndarray:
    i = np.arange(N, dtype=np.int64)
    off = i * (N - 1) - i * (i - 1) // 2
    chunk_items = []
    for g in range(NCHUNKS):
        p0, p1 = g * CPAIRS, (g + 1) * CPAIRS
        lo = int(np.searchsorted(off, p0, side="right")) - 1
        hi = int(np.searchsorted(off, p1, side="left"))
        items = []
        for si in range(lo, hi):
            s0 = max(int(off[si]), p0)
            s1 = min(int(off[si]) + (N - 1 - si), p1)
            if s1 <= s0:
                continue
            j0 = si + 1 + (s0 - int(off[si]))
            ps = 2 * (s0 - p0)
            pe = 2 * (s1 - p0)
            nv = (pe - ps + L - 1) // L
            items.append((si, j0, ps, pe, nv))
        assert len(items) <= MAXIT, len(items)
        chunk_items.append(items)
    order = sorted(range(NCHUNKS), key=lambda g: -len(chunk_items[g]))
    loads = [0] * NW
    counts = [0] * NW
    assign = [[] for _ in range(NW)]
    for g in order:
        w = min((u for u in range(NW) if counts[u] < CPW), key=lambda u: loads[u])
        assign[w].append(g)
        counts[w] += 1
        loads[w] += CHUNK // L + 40 * len(chunk_items[g])
    tbl = np.zeros((NW * CPW, ROWW), dtype=np.int32)
    for w in range(NW):
        for c, g in enumerate(assign[w]):
            row = tbl[w * CPW + c]
            items = chunk_items[g]
            row[0] = len(items)
            row[1] = g * CHUNK
            for t, rec in enumerate(items):
                row[RECW * (t + 1):RECW * (t + 1) + 5] = rec
    return tbl


_TABLE = _build_table()


@functools.cache
def _get_pairs_kernel():
    mesh = plsc.VectorSubcoreMesh(
        core_axis_name="c", subcore_axis_name="s", num_cores=NC, num_subcores=NS
    )
    return functools.partial(
        pl.kernel,
        out_type=jax.ShapeDtypeStruct((FLAT,), jnp.float32),
        mesh=mesh,
        scratch_types=[
            pltpu.VMEM((XPAD,), jnp.float32),
            pltpu.VMEM((ROWW,), jnp.int32),
            pltpu.VMEM((CHUNK + L,), jnp.float32),
            pltpu.VMEM((CHUNK + L,), jnp.float32),
            pltpu.SemaphoreType.DMA,
            pltpu.SemaphoreType.DMA,
        ],
        compiler_params=pltpu.CompilerParams(needs_layout_passes=False),
    )(_pairs_body)


def _pairs_body(x_hbm, tbl_hbm, out_hbm, xv, tblv, buf0, buf1, sem0, sem1):
    w = lax.axis_index("s") * NC + lax.axis_index("c")
    pltpu.sync_copy(x_hbm, xv.at[pl.ds(0, N)])
    iota = lax.iota(jnp.int32, L)
    even = (iota & 1) == 0
    halfi = iota >> 1
    step = jnp.where(even, 0, L // 2)
    bufs = (buf0, buf1)
    sems = (sem0, sem1)
    descs = [None, None]
    for c in range(CPW):
        pltpu.sync_copy(tbl_hbm.at[w * CPW + c], tblv)
        hdr = tblv[pl.ds(0, RECW)]
        nit = hdr[0]
        ooff = pl.multiple_of(hdr[1], 8)
        buf = bufs[c & 1]
        if descs[c & 1] is not None:
            descs[c & 1].wait()

        def item_body(k, carry, buf=buf):
            rec = tblv[pl.ds((k + 1) * RECW, RECW)]
            i_s = rec[0]
            j0 = rec[1]
            ps = rec[2]
            pe = rec[3]
            nv = rec[4]
            idx0 = jnp.where(even, i_s, j0 + halfi)
            pv0 = ps + iota

            def vec_body(_, st):
                idx, pv = st
                val = plsc.load_gather(xv, [idx])
                plsc.store_scatter(buf, [pv], val, mask=pv < pe)
                return (idx + step, pv + L)

            lax.fori_loop(0, nv, vec_body, (idx0, pv0))
            return carry

        lax.fori_loop(0, nit, item_body, jnp.int32(0))
        descs[c & 1] = pltpu.async_copy(
            buf.at[pl.ds(0, CHUNK)], out_hbm.at[pl.ds(ooff, CHUNK)], sems[c & 1]
        )
    descs[0].wait()
    descs[1].wait()


def kernel(x):
    flat = _get_pairs_kernel()(x.reshape(-1), jnp.asarray(_TABLE))
    return flat.reshape(NPAIRS, 2)


# speedup vs baseline: 1688.8788x; 47.0834x over previous
import functools

import numpy as np
import jax
import jax.numpy as jnp
from jax import lax
from jax.experimental import pallas as pl
from jax.experimental.pallas import tpu as pltpu
from jax.experimental.pallas import tpu_sc as plsc

N = 4096
NPAIRS = N * (N - 1) // 2
FLAT = 2 * NPAIRS
NB = NPAIRS // 128
NC, NS, L = 2, 16, 16
NW_ACT = 30
CPW = 13
NCHUNKS = NW_ACT * CPW
CBLOCKS = NB // NCHUNKS
CPAIRS = CBLOCKS * 128
CHUNK = 2 * CPAIRS
RECW = 16
MAXIT = 224
ROWW = (1 + MAXIT) * RECW
XPAD = N + L
BUFPAD = 256 + L


def _build_table() -> np.ndarray:
    i = np.arange(N, dtype=np.int64)
    off = i * (N - 1) - i * (i - 1) // 2
    chunk_items = []
    maxit = 0
    for g in range(NCHUNKS):
        p0, p1 = g * CPAIRS, (g + 1) * CPAIRS
        lo = int(np.searchsorted(off, p0, side="right")) - 1
        hi = int(np.searchsorted(off, p1, side="left"))
        items = []
        for si in range(lo, hi):
            s0 = max(int(off[si]), p0)
            s1 = min(int(off[si]) + (N - 1 - si), p1)
            if s1 <= s0:
                continue
            j0 = si + 1 + (s0 - int(off[si]))
            q0 = s0 - p0
            q1 = s1 - p0
            nv = (q1 - q0 + L - 1) // L
            items.append((si, j0, q0, q1, nv))
        maxit = max(maxit, len(items))
        assert len(items) <= MAXIT, len(items)
        chunk_items.append(items)
    order = sorted(range(NCHUNKS), key=lambda g: -len(chunk_items[g]))
    loads = [0] * NW_ACT
    counts = [0] * NW_ACT
    assign = [[] for _ in range(NW_ACT)]
    for g in order:
        w = min((u for u in range(NW_ACT) if counts[u] < CPW), key=lambda u: loads[u])
        assign[w].append(g)
        counts[w] += 1
        loads[w] += CPAIRS // L + 40 * len(chunk_items[g])
    tbl = np.zeros((NW_ACT * CPW, ROWW), dtype=np.int32)
    for w in range(NW_ACT):
        for c, g in enumerate(assign[w]):
            row = tbl[w * CPW + c]
            items = chunk_items[g]
            row[0] = len(items)
            row[1] = g * CHUNK
            for t, rec in enumerate(items):
                row[RECW * (t + 1):RECW * (t + 1) + 5] = rec
    return tbl


_TABLE = _build_table()


@functools.cache
def _get_pairs_kernel():
    mesh = plsc.VectorSubcoreMesh(
        core_axis_name="c", subcore_axis_name="s", num_cores=NC, num_subcores=NS
    )
    return functools.partial(
        pl.kernel,
        out_type=jax.ShapeDtypeStruct((FLAT,), jnp.float32),
        mesh=mesh,
        scratch_types=[
            pltpu.VMEM((XPAD,), jnp.float32),
            pltpu.VMEM((ROWW,), jnp.int32),
            pltpu.VMEM((CHUNK + BUFPAD,), jnp.float32),
            pltpu.VMEM((CHUNK + BUFPAD,), jnp.float32),
            pltpu.SemaphoreType.DMA,
            pltpu.SemaphoreType.DMA,
        ],
        compiler_params=pltpu.CompilerParams(needs_layout_passes=False),
    )(_pairs_body)


def _pairs_body(x_hbm, tbl_hbm, out_hbm, xv, tblv, buf0, buf1, sem0, sem1):
    w = lax.axis_index("s") * NC + lax.axis_index("c")

    @pl.when(w < NW_ACT)
    def _run():
        pltpu.sync_copy(x_hbm, xv.at[pl.ds(0, N)])
        iota = lax.iota(jnp.int32, L)
        bufs = (buf0, buf1)
        sems = (sem0, sem1)
        descs = [None, None]
        for c in range(CPW):
            pltpu.sync_copy(tbl_hbm.at[w * CPW + c], tblv)
            hdr = tblv[pl.ds(0, RECW)]
            nit = hdr[0]
            ooff = pl.multiple_of(hdr[1], 8)
            buf = bufs[c & 1]
            if descs[c & 1] is not None:
                descs[c & 1].wait()

            def item_body(k, carry, buf=buf):
                rec = tblv[pl.ds((k + 1) * RECW, RECW)]
                i_s = rec[0]
                j0 = rec[1]
                q0 = rec[2]
                q1 = rec[3]
                nv = rec[4]
                xi = plsc.load_gather(xv, [jnp.full((L,), i_s, jnp.int32)])
                jidx0 = j0 + iota
                pv0 = q0 + iota

                def vec_body(_, st):
                    jidx, pv = st
                    pos = ((pv >> 7) << 8) + (pv & 127)
                    mask = pv < q1
                    jval = plsc.load_gather(xv, [jidx])
                    plsc.store_scatter(buf, [pos], xi, mask=mask)
                    plsc.store_scatter(buf, [pos + 128], jval, mask=mask)
                    return (jidx + L, pv + L)

                lax.fori_loop(0, nv, vec_body, (jidx0, pv0))
                return carry

            lax.fori_loop(0, nit, item_body, jnp.int32(0))
            descs[c & 1] = pltpu.async_copy(
                buf.at[pl.ds(0, CHUNK)], out_hbm.at[pl.ds(ooff, CHUNK)], sems[c & 1]
            )
        descs[0].wait()
        descs[1].wait()


def kernel(x):
    flat = _get_pairs_kernel()(x.reshape(-1), jnp.asarray(_TABLE))
    return flat.reshape(NB, 2, 128).swapaxes(1, 2).reshape(NPAIRS, 2)


# speedup vs baseline: 2528.4268x; 1.4971x over previous
import functools

import numpy as np
import jax
import jax.numpy as jnp
from jax import lax
from jax.experimental import pallas as pl
from jax.experimental.pallas import tpu as pltpu
from jax.experimental.pallas import tpu_sc as plsc

N = 4096
NPAIRS = N * (N - 1) // 2
FLAT = 2 * NPAIRS
NB = NPAIRS // 128
NC, NS, L = 2, 16, 16
NW_ACT = 30
CPW = 13
NCHUNKS = NW_ACT * CPW
CBLOCKS = NB // NCHUNKS
CPAIRS = CBLOCKS * 128
CHUNK = 2 * CPAIRS
RECW = 16
MAXIT = 224
ROWW = (1 + MAXIT) * RECW
XPAD = N + L
BUFPAD = 256 + L


def _build_table() -> np.ndarray:
    i = np.arange(N, dtype=np.int64)
    off = i * (N - 1) - i * (i - 1) // 2
    chunk_items = []
    maxit = 0
    for g in range(NCHUNKS):
        p0, p1 = g * CPAIRS, (g + 1) * CPAIRS
        lo = int(np.searchsorted(off, p0, side="right")) - 1
        hi = int(np.searchsorted(off, p1, side="left"))
        items = []
        for si in range(lo, hi):
            s0 = max(int(off[si]), p0)
            s1 = min(int(off[si]) + (N - 1 - si), p1)
            if s1 <= s0:
                continue
            j0 = si + 1 + (s0 - int(off[si]))
            q0 = s0 - p0
            q1 = s1 - p0
            nv16 = ((q1 - q0 + L - 1) // L) * L
            items.append((si, j0, q0, q1, nv16))
        maxit = max(maxit, len(items))
        assert len(items) <= MAXIT, len(items)
        chunk_items.append(items)
    order = sorted(range(NCHUNKS), key=lambda g: -len(chunk_items[g]))
    loads = [0] * NW_ACT
    counts = [0] * NW_ACT
    assign = [[] for _ in range(NW_ACT)]
    for g in order:
        w = min((u for u in range(NW_ACT) if counts[u] < CPW), key=lambda u: loads[u])
        assign[w].append(g)
        counts[w] += 1
        loads[w] += CPAIRS // L + 40 * len(chunk_items[g])
    tbl = np.zeros((NW_ACT * CPW, ROWW), dtype=np.int32)
    for w in range(NW_ACT):
        for c, g in enumerate(assign[w]):
            row = tbl[w * CPW + c]
            items = chunk_items[g]
            row[0] = len(items)
            row[1] = g * CHUNK
            for t, rec in enumerate(items):
                row[RECW * (t + 1):RECW * (t + 1) + 5] = rec
    return tbl


_TABLE = _build_table()


@functools.cache
def _get_pairs_kernel():
    mesh = plsc.VectorSubcoreMesh(
        core_axis_name="c", subcore_axis_name="s", num_cores=NC, num_subcores=NS
    )
    return functools.partial(
        pl.kernel,
        out_type=jax.ShapeDtypeStruct((FLAT,), jnp.float32),
        mesh=mesh,
        scratch_types=[
            pltpu.VMEM((XPAD,), jnp.float32),
            pltpu.VMEM((ROWW,), jnp.int32),
            pltpu.VMEM((CHUNK + BUFPAD,), jnp.float32),
            pltpu.VMEM((CHUNK + BUFPAD,), jnp.float32),
            pltpu.SemaphoreType.DMA,
            pltpu.SemaphoreType.DMA,
        ],
        compiler_params=pltpu.CompilerParams(needs_layout_passes=False),
    )(_pairs_body)


def _pairs_body(x_hbm, tbl_hbm, out_hbm, xv, tblv, buf0, buf1, sem0, sem1):
    w = lax.axis_index("s") * NC + lax.axis_index("c")

    @pl.when(w < NW_ACT)
    def _run():
        pltpu.sync_copy(x_hbm, xv.at[pl.ds(0, N)])
        iota = lax.iota(jnp.int32, L)
        bufs = (buf0, buf1)
        sems = (sem0, sem1)
        descs = [None, None]
        for c in range(CPW):
            pltpu.sync_copy(tbl_hbm.at[w * CPW + c], tblv)
            hdr = tblv[pl.ds(0, RECW)]
            nit = hdr[0]
            ooff = pl.multiple_of(hdr[1], 8)
            buf = bufs[c & 1]
            if descs[c & 1] is not None:
                descs[c & 1].wait()

            def item_body(k, carry, buf=buf):
                rec = tblv[pl.ds((k + 1) * RECW, RECW)]
                i_s = rec[0]
                j0 = rec[1]
                q0 = rec[2]
                q1 = rec[3]
                nv16 = rec[4]
                xi = plsc.load_gather(xv, [jnp.full((L,), i_s, jnp.int32)])
                dj = j0 - q0

                @plsc.parallel_loop(q0, q0 + nv16, step=L, unroll=8)
                def _vec(t):
                    pv = t + iota
                    pos = ((pv >> 7) << 8) + (pv & 127)
                    mask = pv < q1
                    jval = plsc.load_gather(xv, [pv + dj])
                    plsc.store_scatter(buf, [pos], xi, mask=mask)
                    plsc.store_scatter(buf, [pos + 128], jval, mask=mask)

                return carry

            lax.fori_loop(0, nit, item_body, jnp.int32(0))
            descs[c & 1] = pltpu.async_copy(
                buf.at[pl.ds(0, CHUNK)], out_hbm.at[pl.ds(ooff, CHUNK)], sems[c & 1]
            )
        descs[0].wait()
        descs[1].wait()


def kernel(x):
    flat = _get_pairs_kernel()(x.reshape(-1), jnp.asarray(_TABLE))
    return flat.reshape(NB, 2, 128).swapaxes(1, 2).reshape(NPAIRS, 2)


# speedup vs baseline: 2812.6969x; 1.1124x over previous
import functools

import numpy as np
import jax
import jax.numpy as jnp
from jax import lax
from jax.experimental import pallas as pl
from jax.experimental.pallas import tpu as pltpu
from jax.experimental.pallas import tpu_sc as plsc

N = 4096
NPAIRS = N * (N - 1) // 2
FLAT = 2 * NPAIRS
NB = NPAIRS // 128
NC, NS, L = 2, 16, 16
NW_ACT = 30
CPW = 13
NCHUNKS = NW_ACT * CPW
CBLOCKS = NB // NCHUNKS
CPAIRS = CBLOCKS * 128
CHUNK = 2 * CPAIRS
RECW = 4 * L
MAXIT = 224
XPAD = N + L
BUFPAD = 256 + L


def _build_schedule():
    i = np.arange(N, dtype=np.int64)
    off = i * (N - 1) - i * (i - 1) // 2
    chunk_items = []
    for g in range(NCHUNKS):
        p0, p1 = g * CPAIRS, (g + 1) * CPAIRS
        lo = int(np.searchsorted(off, p0, side="right")) - 1
        hi = int(np.searchsorted(off, p1, side="left"))
        items = []
        for si in range(lo, hi):
            s0 = max(int(off[si]), p0)
            s1 = min(int(off[si]) + (N - 1 - si), p1)
            if s1 <= s0:
                continue
            items.append((si, si + 1 + (s0 - int(off[si])), s0 - p0, s1 - p0))
        assert len(items) <= MAXIT, len(items)
        chunk_items.append(items)
    order = sorted(range(NCHUNKS), key=lambda g: -len(chunk_items[g]))
    loads = [0] * NW_ACT
    counts = [0] * NW_ACT
    assign = [[] for _ in range(NW_ACT)]
    for g in order:
        w = min((u for u in range(NW_ACT) if counts[u] < CPW), key=lambda u: loads[u])
        assign[w].append(g)
        counts[w] += 1
        loads[w] += CPAIRS // L + 40 * len(chunk_items[g])
    headers = np.zeros((NW_ACT * L, L), dtype=np.int32)
    recs = []
    for w in range(NW_ACT):
        for c, g in enumerate(assign[w]):
            while len(recs) % 8:
                recs.append(np.zeros(RECW, dtype=np.int32))
            items = chunk_items[g]
            headers[w * L + c, :3] = (len(items), g * CHUNK, len(recs) * RECW)
            for (si, j0, q0, q1) in items:
                rec = np.zeros(RECW, dtype=np.int32)
                rec[0] = q0
                rec[1] = q0 + ((q1 - q0 + L - 1) // L) * L
                rec[L:2 * L] = si
                rec[2 * L:3 * L] = j0 - q0
                rec[3 * L:4 * L] = q1
                recs.append(rec)
    pad = [np.zeros(RECW, dtype=np.int32)] * MAXIT
    return headers.reshape(-1), np.stack(recs + pad).reshape(-1)


_HEADERS, _ITEMS = _build_schedule()


@functools.cache
def _get_pairs_kernel():
    mesh = plsc.VectorSubcoreMesh(
        core_axis_name="c", subcore_axis_name="s", num_cores=NC, num_subcores=NS
    )
    return functools.partial(
        pl.kernel,
        out_type=jax.ShapeDtypeStruct((FLAT,), jnp.float32),
        mesh=mesh,
        scratch_types=[
            pltpu.VMEM((XPAD,), jnp.float32),
            pltpu.VMEM((CPW * L,), jnp.int32),
            pltpu.VMEM((MAXIT * RECW,), jnp.int32),
            pltpu.VMEM((MAXIT * RECW,), jnp.int32),
            pltpu.VMEM((CHUNK + BUFPAD,), jnp.float32),
            pltpu.VMEM((CHUNK + BUFPAD,), jnp.float32),
            pltpu.SemaphoreType.DMA,
            pltpu.SemaphoreType.DMA,
            pltpu.SemaphoreType.DMA,
            pltpu.SemaphoreType.DMA,
        ],
        compiler_params=pltpu.CompilerParams(needs_layout_passes=False),
    )(_pairs_body)


def _pairs_body(x_hbm, hdr_hbm, items_hbm, out_hbm,
                xv, hdrv, itv0, itv1, buf0, buf1,
                osem0, osem1, isem0, isem1):
    w = lax.axis_index("s") * NC + lax.axis_index("c")

    @pl.when(w < NW_ACT)
    def _run():
        pltpu.sync_copy(x_hbm, xv.at[pl.ds(0, N)])
        pltpu.sync_copy(hdr_hbm.at[pl.ds(w * (L * L), CPW * L)], hdrv)
        iota = lax.iota(jnp.int32, L)
        bufs = (buf0, buf1)
        itvs = (itv0, itv1)
        osems = (osem0, osem1)
        isems = (isem0, isem1)

        def hdr_fields(c):
            hv = hdrv[pl.ds(c * L, L)]
            return hv[0], pl.multiple_of(hv[1], 8), pl.multiple_of(hv[2], 8)

        def prefetch(c):
            _, _, it_off = hdr_fields(c)
            return pltpu.async_copy(
                items_hbm.at[pl.ds(it_off, MAXIT * RECW)],
                itvs[c & 1],
                isems[c & 1],
            )

        out_descs = [None, None]
        it_descs = [None, None]
        it_descs[0] = prefetch(0)
        for c in range(CPW):
            nit, ooff, _ = hdr_fields(c)
            buf = bufs[c & 1]
            itv = itvs[c & 1]
            it_descs[c & 1].wait()
            if c + 1 < CPW:
                it_descs[(c + 1) & 1] = prefetch(c + 1)
            if out_descs[c & 1] is not None:
                out_descs[c & 1].wait()

            def item_body(k, carry, buf=buf, itv=itv):
                base = k * RECW
                rec0 = itv[pl.ds(base, L)]
                q0 = rec0[0]
                up = rec0[1]
                xi = plsc.load_gather(xv, [itv[pl.ds(base + L, L)]])
                djv = itv[pl.ds(base + 2 * L, L)]
                q1v = itv[pl.ds(base + 3 * L, L)]
                pv0 = q0 + iota

                @plsc.parallel_loop(q0, up, step=L, unroll=8,
                                    carry=(pv0, pv0 + djv))
                def _vec(t, st):
                    pv, jidx = st
                    pos = pv + (pv & -128)
                    mask = pv < q1v
                    jval = plsc.load_gather(xv, [jidx])
                    plsc.store_scatter(buf, [pos], xi, mask=mask)
                    plsc.store_scatter(buf, [pos + 128], jval, mask=mask)
                    return (pv + L, jidx + L)

                return carry

            lax.fori_loop(0, nit, item_body, jnp.int32(0))
            out_descs[c & 1] = pltpu.async_copy(
                buf.at[pl.ds(0, CHUNK)], out_hbm.at[pl.ds(ooff, CHUNK)],
                osems[c & 1],
            )
        out_descs[0].wait()
        out_descs[1].wait()


def kernel(x):
    flat = _get_pairs_kernel()(
        x.reshape(-1), jnp.asarray(_HEADERS), jnp.asarray(_ITEMS)
    )
    return flat.reshape(NB, 2, 128).swapaxes(1, 2).reshape(NPAIRS, 2)


# speedup vs baseline: 2839.4279x; 1.0095x over previous
import functools

import numpy as np
import jax
import jax.numpy as jnp
from jax import lax
from jax.experimental import pallas as pl
from jax.experimental.pallas import tpu as pltpu
from jax.experimental.pallas import tpu_sc as plsc

N = 4096
NPAIRS = N * (N - 1) // 2
FLAT = 2 * NPAIRS
NB = NPAIRS // 128
NC, NS, L = 2, 16, 16
NW_ACT = 30
CPW = 13
NCHUNKS = NW_ACT * CPW
CBLOCKS = NB // NCHUNKS
CPAIRS = CBLOCKS * 128
CHUNK = 2 * CPAIRS
RECW = L
MAXIT = 224
XPAD = N + L
BUFPAD = 256 + L


def _build_schedule():
    i = np.arange(N, dtype=np.int64)
    off = i * (N - 1) - i * (i - 1) // 2
    chunk_items = []
    for g in range(NCHUNKS):
        p0, p1 = g * CPAIRS, (g + 1) * CPAIRS
        lo = int(np.searchsorted(off, p0, side="right")) - 1
        hi = int(np.searchsorted(off, p1, side="left"))
        items = []
        for si in range(lo, hi):
            s0 = max(int(off[si]), p0)
            s1 = min(int(off[si]) + (N - 1 - si), p1)
            if s1 <= s0:
                continue
            items.append((si, si + 1 + (s0 - int(off[si])), s0 - p0, s1 - p0))
        assert len(items) <= MAXIT, len(items)
        chunk_items.append(items)
    order = sorted(range(NCHUNKS), key=lambda g: -len(chunk_items[g]))
    loads = [0] * NW_ACT
    counts = [0] * NW_ACT
    assign = [[] for _ in range(NW_ACT)]
    for g in order:
        w = min((u for u in range(NW_ACT) if counts[u] < CPW), key=lambda u: loads[u])
        assign[w].append(g)
        counts[w] += 1
        loads[w] += CPAIRS // L + 40 * len(chunk_items[g])
    headers = np.zeros((NW_ACT * L, L), dtype=np.int32)
    recs = []
    for w in range(NW_ACT):
        for c, g in enumerate(assign[w]):
            items = chunk_items[g]
            headers[w * L + c, :3] = (len(items), g * CHUNK, len(recs) * RECW)
            for (si, j0, q0, q1) in items:
                rec = np.zeros(RECW, dtype=np.int32)
                rec[0] = q0
                rec[1] = q0 + ((q1 - q0 + L - 1) // L) * L
                rec[2] = si
                rec[3] = j0 - q0
                rec[4] = q1
                recs.append(rec)
    pad = [np.zeros(RECW, dtype=np.int32)] * MAXIT
    return headers.reshape(-1), np.stack(recs + pad).reshape(-1)


_HEADERS, _ITEMS = _build_schedule()


@functools.cache
def _get_pairs_kernel():
    mesh = plsc.VectorSubcoreMesh(
        core_axis_name="c", subcore_axis_name="s", num_cores=NC, num_subcores=NS
    )
    return functools.partial(
        pl.kernel,
        out_type=jax.ShapeDtypeStruct((FLAT,), jnp.float32),
        mesh=mesh,
        scratch_types=[
            pltpu.VMEM((XPAD,), jnp.float32),
            pltpu.VMEM((CPW * L,), jnp.int32),
            pltpu.VMEM((MAXIT * RECW,), jnp.int32),
            pltpu.VMEM((MAXIT * RECW,), jnp.int32),
            pltpu.VMEM((CHUNK + BUFPAD,), jnp.float32),
            pltpu.VMEM((CHUNK + BUFPAD,), jnp.float32),
            pltpu.SemaphoreType.DMA,
            pltpu.SemaphoreType.DMA,
            pltpu.SemaphoreType.DMA,
            pltpu.SemaphoreType.DMA,
        ],
        compiler_params=pltpu.CompilerParams(needs_layout_passes=False),
    )(_pairs_body)


def _pairs_body(x_hbm, hdr_hbm, items_hbm, out_hbm,
                xv, hdrv, itv0, itv1, buf0, buf1,
                osem0, osem1, isem0, isem1):
    w = lax.axis_index("s") * NC + lax.axis_index("c")

    @pl.when(w < NW_ACT)
    def _run():
        pltpu.sync_copy(x_hbm, xv.at[pl.ds(0, N)])
        pltpu.sync_copy(hdr_hbm.at[pl.ds(w * (L * L), CPW * L)], hdrv)
        iota = lax.iota(jnp.int32, L)
        bufs = (buf0, buf1)
        itvs = (itv0, itv1)
        osems = (osem0, osem1)
        isems = (isem0, isem1)

        def hdr_fields(c):
            hv = hdrv[pl.ds(c * L, L)]
            return hv[0], pl.multiple_of(hv[1], 8), pl.multiple_of(hv[2], 8)

        def prefetch(c):
            _, _, it_off = hdr_fields(c)
            return pltpu.async_copy(
                items_hbm.at[pl.ds(it_off, MAXIT * RECW)],
                itvs[c & 1],
                isems[c & 1],
            )

        out_descs = [None, None]
        it_descs = [None, None]
        it_descs[0] = prefetch(0)
        for c in range(CPW):
            nit, ooff, _ = hdr_fields(c)
            buf = bufs[c & 1]
            itv = itvs[c & 1]
            it_descs[c & 1].wait()
            if c + 1 < CPW:
                it_descs[(c + 1) & 1] = prefetch(c + 1)
            if out_descs[c & 1] is not None:
                out_descs[c & 1].wait()

            def item_body(k, carry, buf=buf, itv=itv):
                rec = itv[pl.ds(k * RECW, L)]
                q0 = rec[0]
                up = rec[1]
                xi = plsc.load_gather(xv, [jnp.full((L,), rec[2])])
                djv = jnp.full((L,), rec[3])
                q1v = jnp.full((L,), rec[4])
                pv0 = q0 + iota

                @plsc.parallel_loop(q0, up, step=L, unroll=8,
                                    carry=(pv0, pv0 + djv))
                def _vec(t, st):
                    pv, jidx = st
                    pos = pv + (pv & -128)
                    mask = pv < q1v
                    jval = plsc.load_gather(xv, [jidx])
                    plsc.store_scatter(buf, [pos], xi, mask=mask)
                    plsc.store_scatter(buf, [pos + 128], jval, mask=mask)
                    return (pv + L, jidx + L)

                return carry

            lax.fori_loop(0, nit, item_body, jnp.int32(0))
            out_descs[c & 1] = pltpu.async_copy(
                buf.at[pl.ds(0, CHUNK)], out_hbm.at[pl.ds(ooff, CHUNK)],
                osems[c & 1],
            )
        out_descs[0].wait()
        out_descs[1].wait()


def kernel(x):
    flat = _get_pairs_kernel()(
        x.reshape(-1), jnp.asarray(_HEADERS), jnp.asarray(_ITEMS)
    )
    return flat.reshape(NB, 2, 128).swapaxes(1, 2).reshape(NPAIRS, 2)


# speedup vs baseline: 3011.8276x; 1.0607x over previous
import functools

import numpy as np
import jax
import jax.numpy as jnp
from jax import lax
from jax.experimental import pallas as pl
from jax.experimental.pallas import tpu as pltpu
from jax.experimental.pallas import tpu_sc as plsc

N = 4096
NPAIRS = N * (N - 1) // 2
FLAT = 2 * NPAIRS
NB = NPAIRS // 128
NC, NS, L = 2, 16, 16
NW_ACT = 30
CPW = 13
NCHUNKS = NW_ACT * CPW
CBLOCKS = NB // NCHUNKS
CPAIRS = CBLOCKS * 128
CHUNK = 2 * CPAIRS
RECW = L
MAXIT = 224
XPAD = N + L
BUFPAD = 256 + L


def _build_schedule():
    i = np.arange(N, dtype=np.int64)
    off = i * (N - 1) - i * (i - 1) // 2
    chunk_items = []
    for g in range(NCHUNKS):
        p0, p1 = g * CPAIRS, (g + 1) * CPAIRS
        lo = int(np.searchsorted(off, p0, side="right")) - 1
        hi = int(np.searchsorted(off, p1, side="left"))
        items = []
        for si in range(lo, hi):
            s0 = max(int(off[si]), p0)
            s1 = min(int(off[si]) + (N - 1 - si), p1)
            if s1 <= s0:
                continue
            items.append((si, si + 1 + (s0 - int(off[si])), s0 - p0, s1 - p0))
        assert len(items) <= MAXIT, len(items)
        chunk_items.append(items)
    order = sorted(range(NCHUNKS), key=lambda g: -len(chunk_items[g]))
    loads = [0] * NW_ACT
    counts = [0] * NW_ACT
    assign = [[] for _ in range(NW_ACT)]
    for g in order:
        w = min((u for u in range(NW_ACT) if counts[u] < CPW), key=lambda u: loads[u])
        assign[w].append(g)
        counts[w] += 1
        loads[w] += CPAIRS // L + 40 * len(chunk_items[g])
    headers = np.zeros((NW_ACT * L, L), dtype=np.int32)
    recs = []
    for w in range(NW_ACT):
        for c, g in enumerate(assign[w]):
            items = chunk_items[g]
            headers[w * L + c, :3] = (len(items), g * CHUNK, len(recs) * RECW)
            for (si, j0, q0, q1) in items:
                rec = np.zeros(RECW, dtype=np.int32)
                rec[0] = q0
                rec[1] = q0 + ((q1 - q0) // L) * L
                rec[2] = si
                rec[3] = j0 - q0
                rec[4] = q1
                recs.append(rec)
    pad = [np.zeros(RECW, dtype=np.int32)] * MAXIT
    return headers.reshape(-1), np.stack(recs + pad).reshape(-1)


_HEADERS, _ITEMS = _build_schedule()


@functools.cache
def _get_pairs_kernel():
    mesh = plsc.VectorSubcoreMesh(
        core_axis_name="c", subcore_axis_name="s", num_cores=NC, num_subcores=NS
    )
    return functools.partial(
        pl.kernel,
        out_type=jax.ShapeDtypeStruct((FLAT,), jnp.float32),
        mesh=mesh,
        scratch_types=[
            pltpu.VMEM((XPAD,), jnp.float32),
            pltpu.VMEM((CPW * L,), jnp.int32),
            pltpu.VMEM((MAXIT * RECW,), jnp.int32),
            pltpu.VMEM((MAXIT * RECW,), jnp.int32),
            pltpu.VMEM((CHUNK + BUFPAD,), jnp.float32),
            pltpu.VMEM((CHUNK + BUFPAD,), jnp.float32),
            pltpu.SemaphoreType.DMA,
            pltpu.SemaphoreType.DMA,
            pltpu.SemaphoreType.DMA,
            pltpu.SemaphoreType.DMA,
        ],
        compiler_params=pltpu.CompilerParams(needs_layout_passes=False),
    )(_pairs_body)


def _pairs_body(x_hbm, hdr_hbm, items_hbm, out_hbm,
                xv, hdrv, itv0, itv1, buf0, buf1,
                osem0, osem1, isem0, isem1):
    w = lax.axis_index("s") * NC + lax.axis_index("c")

    @pl.when(w < NW_ACT)
    def _run():
        pltpu.sync_copy(x_hbm, xv.at[pl.ds(0, N)])
        pltpu.sync_copy(hdr_hbm.at[pl.ds(w * (L * L), CPW * L)], hdrv)
        iota = lax.iota(jnp.int32, L)
        bufs = (buf0, buf1)
        itvs = (itv0, itv1)
        osems = (osem0, osem1)
        isems = (isem0, isem1)

        def hdr_fields(c):
            hv = hdrv[pl.ds(c * L, L)]
            return hv[0], pl.multiple_of(hv[1], 8), pl.multiple_of(hv[2], 8)

        def prefetch(c):
            _, _, it_off = hdr_fields(c)
            return pltpu.async_copy(
                items_hbm.at[pl.ds(it_off, MAXIT * RECW)],
                itvs[c & 1],
                isems[c & 1],
            )

        out_descs = [None, None]
        it_descs = [None, None]
        it_descs[0] = prefetch(0)
        for c in range(CPW):
            nit, ooff, _ = hdr_fields(c)
            buf = bufs[c & 1]
            itv = itvs[c & 1]
            it_descs[c & 1].wait()
            if c + 1 < CPW:
                it_descs[(c + 1) & 1] = prefetch(c + 1)
            if out_descs[c & 1] is not None:
                out_descs[c & 1].wait()

            def item_body(k, carry, buf=buf, itv=itv):
                rec = itv[pl.ds(k * RECW, L)]
                q0 = rec[0]
                fe = rec[1]
                q1 = rec[4]
                xi = plsc.load_gather(xv, [jnp.full((L,), rec[2])])
                djv = jnp.full((L,), rec[3])
                pv0 = q0 + iota

                @plsc.parallel_loop(q0, fe, step=L, unroll=8,
                                    carry=(pv0, pv0 + djv))
                def _vec(t, st):
                    pv, jidx = st
                    pos = pv + (pv & -128)
                    jval = plsc.load_gather(xv, [jidx])
                    plsc.store_scatter(buf, [pos], xi)
                    plsc.store_scatter(buf, [pos + 128], jval)
                    return (pv + L, jidx + L)

                @pl.when(fe < q1)
                def _tail():
                    pv = fe + iota
                    pos = pv + (pv & -128)
                    mask = pv < jnp.full((L,), q1)
                    jval = plsc.load_gather(xv, [pv + djv])
                    plsc.store_scatter(buf, [pos], xi, mask=mask)
                    plsc.store_scatter(buf, [pos + 128], jval, mask=mask)

                return carry

            lax.fori_loop(0, nit, item_body, jnp.int32(0))
            out_descs[c & 1] = pltpu.async_copy(
                buf.at[pl.ds(0, CHUNK)], out_hbm.at[pl.ds(ooff, CHUNK)],
                osems[c & 1],
            )
        out_descs[0].wait()
        out_descs[1].wait()


def kernel(x):
    flat = _get_pairs_kernel()(
        x.reshape(-1), jnp.asarray(_HEADERS), jnp.asarray(_ITEMS)
    )
    return flat.reshape(NB, 2, 128).swapaxes(1, 2).reshape(NPAIRS, 2)
